# baseline (speedup 1.0000x reference)
import functools

import jax
import jax.numpy as jnp
from jax import lax
from jax.experimental import pallas as pl
from jax.experimental.pallas import tpu as pltpu
from jax.experimental.pallas import tpu_sc as plsc

BATCH = 16384
D = 64
NC, NS = 2, 16
NW = NC * NS
BPW = BATCH // NW
CHUNK = 256
NROUND = BPW // CHUNK
L = 16


def _sc_gather(user_ids, item_ids, user_table, item_table):
    mesh = plsc.VectorSubcoreMesh(core_axis_name="c", subcore_axis_name="s")

    @functools.partial(
        pl.kernel,
        out_type=(
            jax.ShapeDtypeStruct((BATCH, D), jnp.float32),
            jax.ShapeDtypeStruct((BATCH, D), jnp.float32),
        ),
        mesh=mesh,
        scratch_types=[
            pltpu.VMEM((BPW,), jnp.int32),
            pltpu.VMEM((BPW,), jnp.int32),
            pltpu.VMEM((CHUNK, D), jnp.float32),
            pltpu.VMEM((CHUNK, D), jnp.float32),
            pltpu.SemaphoreType.DMA,
            pltpu.SemaphoreType.DMA,
        ],
    )
    def gather_kernel(uid_hbm, iid_hbm, ut_hbm, it_hbm, uout_hbm, iout_hbm,
                      uids_v, iids_v, urows, irows, usem, isem):
        wid = lax.axis_index("s") * NC + lax.axis_index("c")
        base = wid * BPW
        pltpu.sync_copy(uid_hbm.at[pl.ds(base, BPW)], uids_v)
        pltpu.sync_copy(iid_hbm.at[pl.ds(base, BPW)], iids_v)
        iota = lax.iota(jnp.int32, L)

        for h in range(NROUND):
            off = h * CHUNK

            def issue(g, _):
                uvec = uids_v[pl.ds(off + g * L, L)]
                ivec = iids_v[pl.ds(off + g * L, L)]
                gL = g * L
                for l in range(L):
                    pltpu.async_copy(ut_hbm.at[pl.ds(uvec[l], 1)],
                                     urows.at[pl.ds(gL + l, 1)], usem)
                    pltpu.async_copy(it_hbm.at[pl.ds(ivec[l], 1)],
                                     irows.at[pl.ds(gL + l, 1)], isem)
                return 0

            lax.fori_loop(0, CHUNK // L, issue, 0)
            pltpu.make_async_copy(ut_hbm.at[pl.ds(0, CHUNK)], urows,
                                  usem).wait()
            pltpu.make_async_copy(it_hbm.at[pl.ds(0, CHUNK)], irows,
                                  isem).wait()
            pltpu.sync_copy(urows, uout_hbm.at[pl.ds(base + off, CHUNK)])
            pltpu.sync_copy(irows, iout_hbm.at[pl.ds(base + off, CHUNK)])

    return gather_kernel(user_ids, item_ids, user_table, item_table)


def _tc_mlp(user_emb, item_emb, content, W_content, b_content, W1, b1, W2, b2):
    BLK = 2048
    cdim = content.shape[1]

    def body(ue_ref, ie_ref, c_ref, wc_ref, bc_ref, w1_ref, b1_ref, w2_ref,
             b2_ref, o_ref):
        c_emb = jnp.dot(c_ref[...], wc_ref[...],
                        preferred_element_type=jnp.float32) + bc_ref[...]
        h = jnp.dot(ue_ref[...], w1_ref[0:D, :],
                    preferred_element_type=jnp.float32)
        h = h + jnp.dot(ie_ref[...], w1_ref[D:2 * D, :],
                        preferred_element_type=jnp.float32)
        h = h + jnp.dot(c_emb, w1_ref[2 * D:3 * D, :],
                        preferred_element_type=jnp.float32)
        h = jnp.maximum(h + b1_ref[...], 0.0)
        o_ref[...] = jnp.dot(h, w2_ref[...],
                             preferred_element_type=jnp.float32) + b2_ref[...]

    full = lambda shape: pl.BlockSpec(shape, lambda i: (0, 0))
    out = pl.pallas_call(
        body,
        grid=(BATCH // BLK,),
        in_specs=[
            pl.BlockSpec((BLK, D), lambda i: (i, 0)),
            pl.BlockSpec((BLK, D), lambda i: (i, 0)),
            pl.BlockSpec((BLK, cdim), lambda i: (i, 0)),
            full((cdim, D)),
            full((1, D)),
            full((3 * D, D)),
            full((1, D)),
            full((D, 1)),
            full((1, 1)),
        ],
        out_specs=pl.BlockSpec((BLK, 1), lambda i: (i, 0)),
        out_shape=jax.ShapeDtypeStruct((BATCH, 1), jnp.float32),
    )(user_emb, item_emb, content, W_content, b_content, W1, b1, W2, b2)
    return out


def kernel(user_ids, item_ids, content, user_table, item_table, W_content,
           b_content, W1, b1, W2, b2):
    user_emb, item_emb = _sc_gather(
        user_ids.astype(jnp.int32), item_ids.astype(jnp.int32),
        user_table, item_table)
    out = _tc_mlp(user_emb, item_emb, content, W_content,
                  b_content.reshape(1, D), W1, b1.reshape(1, D), W2,
                  b2.reshape(1, 1))
    return out.reshape(-1)


# speedup vs baseline: 2.3273x; 2.3273x over previous
import functools

import jax
import jax.numpy as jnp
from jax import lax
from jax.experimental import pallas as pl
from jax.experimental.pallas import tpu as pltpu
from jax.experimental.pallas import tpu_sc as plsc

BATCH = 16384
D = 64
NROWS = 1000000
NC, NS = 2, 16
NW = NC * NS
L = 16
W = 512
NWIN = (NROWS + W - 1) // W
NPAIR = (NWIN // NW + 1 + 1) // 2
HITCAP = BATCH + L
RB = 16


def _sc_scan_gather(user_ids, item_ids, ut_t, it_t, widtab):
    mesh = plsc.VectorSubcoreMesh(core_axis_name="c", subcore_axis_name="s")

    @functools.partial(
        pl.kernel,
        out_type=(
            jax.ShapeDtypeStruct((BATCH * D,), jnp.float32),
            jax.ShapeDtypeStruct((BATCH * D,), jnp.float32),
        ),
        mesh=mesh,
        scratch_types=[
            pltpu.VMEM((BATCH,), jnp.int32),
            pltpu.VMEM((HITCAP,), jnp.int32),
            pltpu.VMEM((HITCAP,), jnp.int32),
            pltpu.VMEM((D, W), jnp.float32),
            pltpu.VMEM((D, W), jnp.float32),
            pltpu.VMEM((2 * L,), jnp.int32),
            pltpu.VMEM((2 * L,), jnp.int32),
            pltpu.VMEM((L,), jnp.int32),
            pltpu.VMEM((L,), jnp.int32),
            pltpu.VMEM((RB * D,), jnp.float32),
            pltpu.SemaphoreType.DMA,
            pltpu.SemaphoreType.DMA,
            pltpu.SemaphoreType.DMA,
        ],
        compiler_params=pltpu.CompilerParams(needs_layout_passes=False),
    )
    def gather_kernel(uid_hbm, iid_hbm, ut_hbm, it_hbm, wtab_hbm,
                      uout_hbm, iout_hbm,
                      ids_v, hitc, hitp, pA, pB, wc, wp, widb, tmpv, ring,
                      semA, semB, osem):
        wid = lax.axis_index("s") * NC + lax.axis_index("c")
        iota = lax.iota(jnp.int32, L)
        fifteen = iota * 0 + (L - 1)
        nwin_w = NWIN // NW + lax.max(0, lax.min(1, (NWIN % NW) - wid))
        pltpu.sync_copy(wtab_hbm.at[pl.ds(wid * L, L)], widb)
        widv = widb[pl.ds(0, L)]

        def drain_one():
            pltpu.make_async_copy(ut_hbm.at[0, pl.ds(0, D)],
                                  ring.at[pl.ds(0, D)], osem).wait()

        for id_hbm, t_hbm, out_hbm in ((uid_hbm, ut_hbm, uout_hbm),
                                       (iid_hbm, it_hbm, iout_hbm)):
            pltpu.sync_copy(id_hbm, ids_v)

            def comp_chunk(g, carry):
                offv, posv = carry
                vec = ids_v[pl.ds(g * L, L)]
                m = ((vec >> 9) & (NW - 1)) == widv
                mi = jnp.where(m, iota * 0 + 1, iota * 0)
                incl = plsc.cumsum(mi)
                idx = jnp.where(m, offv + incl - mi, BATCH + iota)
                plsc.store_scatter(hitc, [idx], vec)
                plsc.store_scatter(hitp, [idx], posv)
                tmpv[pl.ds(0, L)] = incl
                cntv = plsc.load_gather(tmpv, [fifteen])
                return (offv + cntv, posv + L)

            offv, _ = lax.fori_loop(0, BATCH // L, comp_chunk,
                                    (iota * 0, iota))
            nhits = offv[0]
            hitc[pl.ds(nhits, L)] = jnp.zeros((L,), jnp.int32) - 1
            nch = (nhits + L - 1) // L

            def fetch(win, panel, sem):
                cw = pl.multiple_of(win * W, 128)
                pltpu.async_copy(t_hbm.at[:, pl.ds(cw, W)], panel, sem)

            def wait_panel(panel, sem):
                pltpu.make_async_copy(t_hbm.at[:, pl.ds(0, W)], panel,
                                      sem).wait()

            def process(winv, panel, gc0):
                winoff = winv * W

                def scan_chunk(c, gc):
                    colv = hitc[pl.ds(c * L, L)]
                    posv = hitp[pl.ds(c * L, L)]
                    d = colv - winoff
                    m = (d >= 0) & (d < W)
                    mi = jnp.where(m, iota * 0 + 1, iota * 0)
                    incl = plsc.cumsum(mi)
                    cnt = incl[L - 1]
                    idx = jnp.where(m, incl - mi, L + iota)
                    plsc.store_scatter(wc, [idx], d)
                    plsc.store_scatter(wp, [idx], posv)

                    def extract_one(e, carry):
                        gc2, ev = carry
                        j = wp[pl.ds(e, L)][0]
                        slot = gc2 & (RB - 1)

                        @pl.when(gc2 >= RB)
                        def _():
                            drain_one()

                        csplat = plsc.load_gather(wc, [ev])
                        for kk in range(D // L):
                            vals = plsc.load_gather(
                                panel, [iota + kk * L, csplat])
                            ring[pl.ds(slot * D + kk * L, L)] = vals
                        pltpu.async_copy(
                            ring.at[pl.ds(slot * D, D)],
                            out_hbm.at[pl.ds(j * D, D)], osem)
                        return (gc2 + 1, ev + 1)

                    gc, _ = lax.fori_loop(0, cnt, extract_one,
                                          (gc, iota * 0))
                    return gc

                return lax.fori_loop(0, nch, scan_chunk, gc0)

            fetch(wid, pA, semA)

            def pair_body(k2, carry):
                gc, winAv = carry
                j0 = 2 * k2
                j1 = j0 + 1
                j2 = j0 + 2
                win1 = wid + NW * j1
                win2 = wid + NW * j2
                winBv = winAv + NW

                @pl.when(j1 < nwin_w)
                def _():
                    fetch(win1, pB, semB)

                wait_panel(pA, semA)
                gc = process(winAv, pA, gc)

                @pl.when(j2 < nwin_w)
                def _():
                    fetch(win2, pA, semA)

                def do_b(gcb):
                    wait_panel(pB, semB)
                    return process(winBv, pB, gcb)

                gc = lax.cond(j1 < nwin_w, do_b, lambda g: g, gc)
                return (gc, winAv + 2 * NW)

            gcnt, _ = lax.fori_loop(0, NPAIR, pair_body, (0, widv))
            rem = lax.min(gcnt, RB)
            lax.fori_loop(0, rem, lambda e, x: (drain_one(), x)[1], 0)

    return gather_kernel(user_ids, item_ids, ut_t, it_t, widtab)


def _tc_mlp(user_emb, item_emb, content, W_content, b_content, W1, b1, W2, b2):
    BLK = 2048
    cdim = content.shape[1]

    def body(ue_ref, ie_ref, c_ref, wc_ref, bc_ref, w1_ref, b1_ref, w2_ref,
             b2_ref, o_ref):
        c_emb = jnp.dot(c_ref[...], wc_ref[...],
                        preferred_element_type=jnp.float32) + bc_ref[...]
        h = jnp.dot(ue_ref[...], w1_ref[0:D, :],
                    preferred_element_type=jnp.float32)
        h = h + jnp.dot(ie_ref[...], w1_ref[D:2 * D, :],
                        preferred_element_type=jnp.float32)
        h = h + jnp.dot(c_emb, w1_ref[2 * D:3 * D, :],
                        preferred_element_type=jnp.float32)
        h = jnp.maximum(h + b1_ref[...], 0.0)
        o_ref[...] = jnp.dot(h, w2_ref[...],
                             preferred_element_type=jnp.float32) + b2_ref[...]

    full = lambda shape: pl.BlockSpec(shape, lambda i: (0, 0))
    out = pl.pallas_call(
        body,
        grid=(BATCH // BLK,),
        in_specs=[
            pl.BlockSpec((BLK, D), lambda i: (i, 0)),
            pl.BlockSpec((BLK, D), lambda i: (i, 0)),
            pl.BlockSpec((BLK, cdim), lambda i: (i, 0)),
            full((cdim, D)),
            full((1, D)),
            full((3 * D, D)),
            full((1, D)),
            full((D, 1)),
            full((1, 1)),
        ],
        out_specs=pl.BlockSpec((BLK, 1), lambda i: (i, 0)),
        out_shape=jax.ShapeDtypeStruct((BATCH, 1), jnp.float32),
    )(user_emb, item_emb, content, W_content, b_content, W1, b1, W2, b2)
    return out


def kernel(user_ids, item_ids, content, user_table, item_table, W_content,
           b_content, W1, b1, W2, b2):
    widtab = jnp.repeat(jnp.arange(NW, dtype=jnp.int32), L)
    uflat, iflat = _sc_scan_gather(
        user_ids.astype(jnp.int32), item_ids.astype(jnp.int32),
        jnp.transpose(user_table), jnp.transpose(item_table), widtab)
    user_emb = uflat.reshape(BATCH, D)
    item_emb = iflat.reshape(BATCH, D)
    out = _tc_mlp(user_emb, item_emb, content, W_content,
                  b_content.reshape(1, D), W1, b1.reshape(1, D), W2,
                  b2.reshape(1, 1))
    return out.reshape(-1)


# speedup vs baseline: 2.4016x; 1.0319x over previous
import functools

import jax
import jax.numpy as jnp
from jax import lax
from jax.experimental import pallas as pl
from jax.experimental.pallas import tpu as pltpu
from jax.experimental.pallas import tpu_sc as plsc

BATCH = 16384
D = 64
NROWS = 1000000
NC, NS = 2, 16
NW = NC * NS
L = 16
W = 512
NWIN = (NROWS + W - 1) // W
NPAIR = (NWIN // NW + 1 + 1) // 2
HITCAP = BATCH + L
RB = 16


def _sc_scan_gather(user_ids, item_ids, ut_t, it_t, widtab):
    mesh = plsc.VectorSubcoreMesh(core_axis_name="c", subcore_axis_name="s")

    @functools.partial(
        pl.kernel,
        out_type=(
            jax.ShapeDtypeStruct((BATCH, D), jnp.float32),
            jax.ShapeDtypeStruct((BATCH, D), jnp.float32),
        ),
        mesh=mesh,
        scratch_types=[
            pltpu.VMEM((BATCH,), jnp.int32),
            pltpu.VMEM((HITCAP,), jnp.int32),
            pltpu.VMEM((HITCAP,), jnp.int32),
            pltpu.VMEM((D, W), jnp.float32),
            pltpu.VMEM((D, W), jnp.float32),
            pltpu.VMEM((2 * L,), jnp.int32),
            pltpu.VMEM((2 * L,), jnp.int32),
            pltpu.VMEM((L,), jnp.int32),
            pltpu.VMEM((L,), jnp.int32),
            pltpu.VMEM((RB, D), jnp.float32),
            pltpu.SemaphoreType.DMA,
            pltpu.SemaphoreType.DMA,
            pltpu.SemaphoreType.DMA,
        ],
        compiler_params=pltpu.CompilerParams(needs_layout_passes=False),
    )
    def gather_kernel(uid_hbm, iid_hbm, ut_hbm, it_hbm, wtab_hbm,
                      uout_hbm, iout_hbm,
                      ids_v, hitc, hitp, pA, pB, wc, wp, widb, tmpv, ring,
                      semA, semB, osem):
        wid = lax.axis_index("s") * NC + lax.axis_index("c")
        iota = lax.iota(jnp.int32, L)
        fifteen = iota * 0 + (L - 1)
        nwin_w = NWIN // NW + lax.max(0, lax.min(1, (NWIN % NW) - wid))
        pltpu.sync_copy(wtab_hbm.at[pl.ds(wid * L, L)], widb)
        widv = widb[pl.ds(0, L)]

        def drain_one():
            pltpu.make_async_copy(uid_hbm.at[pl.ds(0, D)],
                                  ids_v.at[pl.ds(0, D)], osem).wait()

        for id_hbm, t_hbm, out_hbm in ((uid_hbm, ut_hbm, uout_hbm),
                                       (iid_hbm, it_hbm, iout_hbm)):
            pltpu.sync_copy(id_hbm, ids_v)

            def comp_chunk(g, carry):
                offv, posv = carry
                vec = ids_v[pl.ds(g * L, L)]
                m = ((vec >> 9) & (NW - 1)) == widv
                mi = jnp.where(m, iota * 0 + 1, iota * 0)
                incl = plsc.cumsum(mi)
                idx = jnp.where(m, offv + incl - mi, BATCH + iota)
                plsc.store_scatter(hitc, [idx], vec)
                plsc.store_scatter(hitp, [idx], posv)
                tmpv[pl.ds(0, L)] = incl
                cntv = plsc.load_gather(tmpv, [fifteen])
                return (offv + cntv, posv + L)

            offv, _ = lax.fori_loop(0, BATCH // L, comp_chunk,
                                    (iota * 0, iota))
            nhits = offv[0]
            hitc[pl.ds(nhits, L)] = jnp.zeros((L,), jnp.int32) - 1
            nch = (nhits + L - 1) // L

            def fetch(win, panel, sem):
                cw = pl.multiple_of(win * W, 128)
                pltpu.async_copy(t_hbm.at[:, pl.ds(cw, W)], panel, sem)

            def wait_panel(panel, sem):
                pltpu.make_async_copy(t_hbm.at[:, pl.ds(0, W)], panel,
                                      sem).wait()

            def process(winv, panel, gc0):
                winoff = winv * W

                def scan_chunk(c, gc):
                    colv = hitc[pl.ds(c * L, L)]
                    posv = hitp[pl.ds(c * L, L)]
                    d = colv - winoff
                    m = (d >= 0) & (d < W)
                    mi = jnp.where(m, iota * 0 + 1, iota * 0)
                    incl = plsc.cumsum(mi)
                    cnt = incl[L - 1]
                    idx = jnp.where(m, incl - mi, L + iota)
                    plsc.store_scatter(wc, [idx], d)
                    plsc.store_scatter(wp, [idx], posv)

                    def extract_one(e, carry):
                        gc2, ev = carry
                        j = wp[pl.ds(e, L)][0]
                        slot = gc2 & (RB - 1)

                        @pl.when(gc2 >= RB)
                        def _():
                            drain_one()

                        csplat = plsc.load_gather(wc, [ev])
                        for kk in range(D // L):
                            vals = plsc.load_gather(
                                panel, [iota + kk * L, csplat])
                            ring[slot, pl.ds(kk * L, L)] = vals
                        pltpu.async_copy(
                            ring.at[pl.ds(slot, 1)],
                            out_hbm.at[pl.ds(j, 1)], osem)
                        return (gc2 + 1, ev + 1)

                    gc, _ = lax.fori_loop(0, cnt, extract_one,
                                          (gc, iota * 0))
                    return gc

                return lax.fori_loop(0, nch, scan_chunk, gc0)

            fetch(wid, pA, semA)

            def pair_body(k2, carry):
                gc, winAv = carry
                j0 = 2 * k2
                j1 = j0 + 1
                j2 = j0 + 2
                win1 = wid + NW * j1
                win2 = wid + NW * j2
                winBv = winAv + NW

                @pl.when(j1 < nwin_w)
                def _():
                    fetch(win1, pB, semB)

                wait_panel(pA, semA)
                gc = process(winAv, pA, gc)

                @pl.when(j2 < nwin_w)
                def _():
                    fetch(win2, pA, semA)

                def do_b(gcb):
                    wait_panel(pB, semB)
                    return process(winBv, pB, gcb)

                gc = lax.cond(j1 < nwin_w, do_b, lambda g: g, gc)
                return (gc, winAv + 2 * NW)

            gcnt, _ = lax.fori_loop(0, NPAIR, pair_body, (0, widv))
            rem = lax.min(gcnt, RB)
            lax.fori_loop(0, rem, lambda e, x: (drain_one(), x)[1], 0)

    return gather_kernel(user_ids, item_ids, ut_t, it_t, widtab)


def _tc_mlp(user_emb, item_emb, content, W_content, b_content, W1, b1, W2, b2):
    BLK = 2048
    cdim = content.shape[1]

    def body(ue_ref, ie_ref, c_ref, wc_ref, bc_ref, w1_ref, b1_ref, w2_ref,
             b2_ref, o_ref):
        c_emb = jnp.dot(c_ref[...], wc_ref[...],
                        preferred_element_type=jnp.float32) + bc_ref[...]
        h = jnp.dot(ue_ref[...], w1_ref[0:D, :],
                    preferred_element_type=jnp.float32)
        h = h + jnp.dot(ie_ref[...], w1_ref[D:2 * D, :],
                        preferred_element_type=jnp.float32)
        h = h + jnp.dot(c_emb, w1_ref[2 * D:3 * D, :],
                        preferred_element_type=jnp.float32)
        h = jnp.maximum(h + b1_ref[...], 0.0)
        o_ref[...] = jnp.dot(h, w2_ref[...],
                             preferred_element_type=jnp.float32) + b2_ref[...]

    full = lambda shape: pl.BlockSpec(shape, lambda i: (0, 0))
    out = pl.pallas_call(
        body,
        grid=(BATCH // BLK,),
        in_specs=[
            pl.BlockSpec((BLK, D), lambda i: (i, 0)),
            pl.BlockSpec((BLK, D), lambda i: (i, 0)),
            pl.BlockSpec((BLK, cdim), lambda i: (i, 0)),
            full((cdim, D)),
            full((1, D)),
            full((3 * D, D)),
            full((1, D)),
            full((D, 1)),
            full((1, 1)),
        ],
        out_specs=pl.BlockSpec((BLK, 1), lambda i: (i, 0)),
        out_shape=jax.ShapeDtypeStruct((BATCH, 1), jnp.float32),
    )(user_emb, item_emb, content, W_content, b_content, W1, b1, W2, b2)
    return out


def kernel(user_ids, item_ids, content, user_table, item_table, W_content,
           b_content, W1, b1, W2, b2):
    widtab = jnp.repeat(jnp.arange(NW, dtype=jnp.int32), L)
    user_emb, item_emb = _sc_scan_gather(
        user_ids.astype(jnp.int32), item_ids.astype(jnp.int32),
        jnp.transpose(user_table), jnp.transpose(item_table), widtab)
    out = _tc_mlp(user_emb, item_emb, content, W_content,
                  b_content.reshape(1, D), W1, b1.reshape(1, D), W2,
                  b2.reshape(1, 1))
    return out.reshape(-1)


# speedup vs baseline: 2.4162x; 1.0061x over previous
import functools

import jax
import jax.numpy as jnp
from jax import lax
from jax.experimental import pallas as pl
from jax.experimental.pallas import tpu as pltpu
from jax.experimental.pallas import tpu_sc as plsc

BATCH = 16384
D = 64
NROWS = 1000000
NC, NS = 2, 16
NW = NC * NS
L = 16
W = 512
NWIN = (NROWS + W - 1) // W
NPAIR = (NWIN // NW + 1 + 1) // 2
HITCAP = BATCH + L
RB = 16


def _sc_scan_gather(user_ids, item_ids, ut_t, it_t, widtab):
    mesh = plsc.VectorSubcoreMesh(core_axis_name="c", subcore_axis_name="s")

    @functools.partial(
        pl.kernel,
        out_type=(
            jax.ShapeDtypeStruct((BATCH, D), jnp.float32),
            jax.ShapeDtypeStruct((BATCH, D), jnp.float32),
        ),
        mesh=mesh,
        scratch_types=[
            pltpu.VMEM((BATCH,), jnp.int32),
            pltpu.VMEM((HITCAP,), jnp.int32),
            pltpu.VMEM((HITCAP,), jnp.int32),
            pltpu.VMEM((D, W), jnp.float32),
            pltpu.VMEM((D, W), jnp.float32),
            pltpu.VMEM((2 * L,), jnp.int32),
            pltpu.VMEM((2 * L,), jnp.int32),
            pltpu.VMEM((L,), jnp.int32),
            pltpu.VMEM((L,), jnp.int32),
            pltpu.VMEM((RB, D), jnp.float32),
            pltpu.SemaphoreType.DMA,
            pltpu.SemaphoreType.DMA,
            pltpu.SemaphoreType.DMA,
        ],
        compiler_params=pltpu.CompilerParams(needs_layout_passes=False),
    )
    def gather_kernel(uid_hbm, iid_hbm, ut_hbm, it_hbm, wtab_hbm,
                      uout_hbm, iout_hbm,
                      ids_v, hitc, hitp, pA, pB, wc, wp, widb, tmpv, ring,
                      semA, semB, osem):
        wid = lax.axis_index("s") * NC + lax.axis_index("c")
        iota = lax.iota(jnp.int32, L)
        fifteen = iota * 0 + (L - 1)
        nwin_w = NWIN // NW + lax.max(0, lax.min(1, (NWIN % NW) - wid))
        pltpu.sync_copy(wtab_hbm.at[pl.ds(wid * L, L)], widb)
        widv = widb[pl.ds(0, L)]

        def drain_one():
            pltpu.make_async_copy(uid_hbm.at[pl.ds(0, D)],
                                  ids_v.at[pl.ds(0, D)], osem).wait()

        for id_hbm, t_hbm, out_hbm in ((uid_hbm, ut_hbm, uout_hbm),
                                       (iid_hbm, it_hbm, iout_hbm)):
            pltpu.sync_copy(id_hbm, ids_v)

            def comp_chunk(g, carry):
                offv, posv = carry
                vec = ids_v[pl.ds(g * L, L)]
                m = ((vec >> 9) & (NW - 1)) == widv
                mi = jnp.where(m, iota * 0 + 1, iota * 0)
                incl = plsc.cumsum(mi)
                idx = jnp.where(m, offv + incl - mi, BATCH + iota)
                plsc.store_scatter(hitc, [idx], vec)
                plsc.store_scatter(hitp, [idx], posv)
                tmpv[pl.ds(0, L)] = incl
                cntv = plsc.load_gather(tmpv, [fifteen])
                return (offv + cntv, posv + L)

            offv, _ = lax.fori_loop(0, BATCH // L, comp_chunk,
                                    (iota * 0, iota))
            nhits = offv[0]
            hitc[pl.ds(nhits, L)] = jnp.zeros((L,), jnp.int32) - 1
            nch = (nhits + L - 1) // L

            CLAMP = (NROWS + 127) // 128 * 128 - W

            def fetch(win, panel, sem):
                cw = pl.multiple_of(lax.min(win * W, CLAMP), 128)
                pltpu.async_copy(t_hbm.at[:, pl.ds(cw, W)], panel, sem)

            def wait_panel(panel, sem):
                pltpu.make_async_copy(t_hbm.at[:, pl.ds(0, W)], panel,
                                      sem).wait()

            def process(winv, panel, gc0):
                winoff = winv * W

                panel_org = jnp.minimum(winoff, iota * 0 + CLAMP)

                def scan_chunk(c, gc):
                    colv = hitc[pl.ds(c * L, L)]
                    posv = hitp[pl.ds(c * L, L)]
                    d = colv - winoff
                    m = (d >= 0) & (d < W)
                    mi = jnp.where(m, iota * 0 + 1, iota * 0)
                    incl = plsc.cumsum(mi)
                    cnt = incl[L - 1]
                    idx = jnp.where(m, incl - mi, L + iota)
                    plsc.store_scatter(wc, [idx], colv - panel_org)
                    plsc.store_scatter(wp, [idx], posv)

                    def extract_one(e, carry):
                        gc2, ev = carry
                        j = wp[pl.ds(e, L)][0]
                        slot = gc2 & (RB - 1)

                        @pl.when(gc2 >= RB)
                        def _():
                            drain_one()

                        csplat = plsc.load_gather(wc, [ev])
                        for kk in range(D // L):
                            vals = plsc.load_gather(
                                panel, [iota + kk * L, csplat])
                            ring[slot, pl.ds(kk * L, L)] = vals
                        pltpu.async_copy(
                            ring.at[pl.ds(slot, 1)],
                            out_hbm.at[pl.ds(j, 1)], osem)
                        return (gc2 + 1, ev + 1)

                    gc, _ = lax.fori_loop(0, cnt, extract_one,
                                          (gc, iota * 0))
                    return gc

                return lax.fori_loop(0, nch, scan_chunk, gc0)

            fetch(wid, pA, semA)

            def pair_body(k2, carry):
                gc, winAv = carry
                j0 = 2 * k2
                j1 = j0 + 1
                j2 = j0 + 2
                win1 = wid + NW * j1
                win2 = wid + NW * j2
                winBv = winAv + NW

                @pl.when(j1 < nwin_w)
                def _():
                    fetch(win1, pB, semB)

                wait_panel(pA, semA)
                gc = process(winAv, pA, gc)

                @pl.when(j2 < nwin_w)
                def _():
                    fetch(win2, pA, semA)

                def do_b(gcb):
                    wait_panel(pB, semB)
                    return process(winBv, pB, gcb)

                gc = lax.cond(j1 < nwin_w, do_b, lambda g: g, gc)
                return (gc, winAv + 2 * NW)

            gcnt, _ = lax.fori_loop(0, NPAIR, pair_body, (0, widv))
            rem = lax.min(gcnt, RB)
            lax.fori_loop(0, rem, lambda e, x: (drain_one(), x)[1], 0)

    return gather_kernel(user_ids, item_ids, ut_t, it_t, widtab)


def _tc_mlp(user_emb, item_emb, content, W_content, b_content, W1, b1, W2, b2):
    BLK = 2048
    cdim = content.shape[1]

    def body(ue_ref, ie_ref, c_ref, wc_ref, bc_ref, w1_ref, b1_ref, w2_ref,
             b2_ref, o_ref):
        c_emb = jnp.dot(c_ref[...], wc_ref[...],
                        preferred_element_type=jnp.float32) + bc_ref[...]
        h = jnp.dot(ue_ref[...], w1_ref[0:D, :],
                    preferred_element_type=jnp.float32)
        h = h + jnp.dot(ie_ref[...], w1_ref[D:2 * D, :],
                        preferred_element_type=jnp.float32)
        h = h + jnp.dot(c_emb, w1_ref[2 * D:3 * D, :],
                        preferred_element_type=jnp.float32)
        h = jnp.maximum(h + b1_ref[...], 0.0)
        o_ref[...] = jnp.dot(h, w2_ref[...],
                             preferred_element_type=jnp.float32) + b2_ref[...]

    full = lambda shape: pl.BlockSpec(shape, lambda i: (0, 0))
    out = pl.pallas_call(
        body,
        grid=(BATCH // BLK,),
        in_specs=[
            pl.BlockSpec((BLK, D), lambda i: (i, 0)),
            pl.BlockSpec((BLK, D), lambda i: (i, 0)),
            pl.BlockSpec((BLK, cdim), lambda i: (i, 0)),
            full((cdim, D)),
            full((1, D)),
            full((3 * D, D)),
            full((1, D)),
            full((D, 1)),
            full((1, 1)),
        ],
        out_specs=pl.BlockSpec((BLK, 1), lambda i: (i, 0)),
        out_shape=jax.ShapeDtypeStruct((BATCH, 1), jnp.float32),
    )(user_emb, item_emb, content, W_content, b_content, W1, b1, W2, b2)
    return out


def kernel(user_ids, item_ids, content, user_table, item_table, W_content,
           b_content, W1, b1, W2, b2):
    widtab = jnp.repeat(jnp.arange(NW, dtype=jnp.int32), L)
    user_emb, item_emb = _sc_scan_gather(
        user_ids.astype(jnp.int32), item_ids.astype(jnp.int32),
        jnp.transpose(user_table), jnp.transpose(item_table), widtab)
    out = _tc_mlp(user_emb, item_emb, content, W_content,
                  b_content.reshape(1, D), W1, b1.reshape(1, D), W2,
                  b2.reshape(1, 1))
    return out.reshape(-1)


# speedup vs baseline: 2.4659x; 1.0206x over previous
import functools

import jax
import jax.numpy as jnp
from jax import lax
from jax.experimental import pallas as pl
from jax.experimental.pallas import tpu as pltpu
from jax.experimental.pallas import tpu_sc as plsc

BATCH = 16384
D = 64
NROWS = 1000000
NC, NS = 2, 16
NW = NC * NS
L = 16
W = 512
NWIN = (NROWS + W - 1) // W
NPAIR = (NWIN // NW + 1 + 1) // 2
HITCAP = BATCH + L
RB = 16


def _sc_scan_gather(user_ids, item_ids, ut_t, it_t, widtab):
    mesh = plsc.VectorSubcoreMesh(core_axis_name="c", subcore_axis_name="s")

    @functools.partial(
        pl.kernel,
        out_type=(
            jax.ShapeDtypeStruct((BATCH, D), jnp.float32),
            jax.ShapeDtypeStruct((BATCH, D), jnp.float32),
        ),
        mesh=mesh,
        scratch_types=[
            pltpu.VMEM((BATCH,), jnp.int32),
            pltpu.VMEM((HITCAP,), jnp.int32),
            pltpu.VMEM((HITCAP,), jnp.int32),
            pltpu.VMEM((D, W), jnp.float32),
            pltpu.VMEM((D, W), jnp.float32),
            pltpu.VMEM((2 * L,), jnp.int32),
            pltpu.VMEM((2 * L,), jnp.int32),
            pltpu.VMEM((L,), jnp.int32),
            pltpu.VMEM((L,), jnp.int32),
            pltpu.VMEM((RB, D), jnp.float32),
            pltpu.SemaphoreType.DMA,
            pltpu.SemaphoreType.DMA,
            pltpu.SemaphoreType.DMA,
        ],
        compiler_params=pltpu.CompilerParams(needs_layout_passes=False),
    )
    def gather_kernel(uid_hbm, iid_hbm, ut_hbm, it_hbm, wtab_hbm,
                      uout_hbm, iout_hbm,
                      ids_v, hitc, hitp, pA, pB, wc, wp, widb, tmpv, ring,
                      semA, semB, osem):
        wid = lax.axis_index("s") * NC + lax.axis_index("c")
        iota = lax.iota(jnp.int32, L)
        fifteen = iota * 0 + (L - 1)
        nwin_w = NWIN // NW + lax.max(0, lax.min(1, (NWIN % NW) - wid))
        pltpu.sync_copy(wtab_hbm.at[pl.ds(wid * L, L)], widb)
        widv = widb[pl.ds(0, L)]

        def drain_one():
            pltpu.make_async_copy(uid_hbm.at[pl.ds(0, D)],
                                  ids_v.at[pl.ds(0, D)], osem).wait()

        for id_hbm, t_hbm, out_hbm in ((uid_hbm, ut_hbm, uout_hbm),
                                       (iid_hbm, it_hbm, iout_hbm)):
            pltpu.sync_copy(id_hbm, ids_v)

            def fetch0(win, panel, sem):
                cw0 = pl.multiple_of(win * W, 128)
                pltpu.async_copy(t_hbm.at[:, pl.ds(cw0, W)], panel, sem)

            fetch0(wid, pA, semA)

            def comp_chunk(g, carry):
                offv, posv = carry
                vec = ids_v[pl.ds(g * L, L)]
                m = ((vec >> 9) & (NW - 1)) == widv
                mi = jnp.where(m, iota * 0 + 1, iota * 0)
                incl = plsc.cumsum(mi)
                idx = jnp.where(m, offv + incl - mi, BATCH + iota)
                plsc.store_scatter(hitc, [idx], vec)
                plsc.store_scatter(hitp, [idx], posv)
                tmpv[pl.ds(0, L)] = incl
                cntv = plsc.load_gather(tmpv, [fifteen])
                return (offv + cntv, posv + L)

            offv, _ = lax.fori_loop(0, BATCH // L, comp_chunk,
                                    (iota * 0, iota))
            nhits = offv[0]
            hitc[pl.ds(nhits, L)] = jnp.zeros((L,), jnp.int32) - 1
            nch = (nhits + L - 1) // L

            CLAMP = (NROWS + 127) // 128 * 128 - W

            def fetch(win, panel, sem):
                cw = pl.multiple_of(lax.min(win * W, CLAMP), 128)
                pltpu.async_copy(t_hbm.at[:, pl.ds(cw, W)], panel, sem)

            def wait_panel(panel, sem):
                pltpu.make_async_copy(t_hbm.at[:, pl.ds(0, W)], panel,
                                      sem).wait()

            def process(winv, panel, gc0):
                winoff = winv * W

                panel_org = jnp.minimum(winoff, iota * 0 + CLAMP)

                def scan_chunk(c, gc):
                    colv = hitc[pl.ds(c * L, L)]
                    posv = hitp[pl.ds(c * L, L)]
                    d = colv - winoff
                    m = (d >= 0) & (d < W)
                    mi = jnp.where(m, iota * 0 + 1, iota * 0)
                    incl = plsc.cumsum(mi)
                    cnt = incl[L - 1]
                    idx = jnp.where(m, incl - mi, L + iota)
                    plsc.store_scatter(wc, [idx], colv - panel_org)
                    plsc.store_scatter(wp, [idx], posv)

                    def extract_one(e, carry):
                        gc2, ev = carry
                        j = wp[pl.ds(e, L)][0]
                        slot = gc2 & (RB - 1)

                        @pl.when(gc2 >= RB)
                        def _():
                            drain_one()

                        csplat = plsc.load_gather(wc, [ev])
                        for kk in range(D // L):
                            vals = plsc.load_gather(
                                panel, [iota + kk * L, csplat])
                            ring[slot, pl.ds(kk * L, L)] = vals
                        pltpu.async_copy(
                            ring.at[pl.ds(slot, 1)],
                            out_hbm.at[pl.ds(j, 1)], osem)
                        return (gc2 + 1, ev + 1)

                    gc, _ = lax.fori_loop(0, cnt, extract_one,
                                          (gc, iota * 0))
                    return gc

                return lax.fori_loop(0, nch, scan_chunk, gc0)

            def pair_body(k2, carry):
                gc, winAv = carry
                j0 = 2 * k2
                j1 = j0 + 1
                j2 = j0 + 2
                win1 = wid + NW * j1
                win2 = wid + NW * j2
                winBv = winAv + NW

                @pl.when(j1 < nwin_w)
                def _():
                    fetch(win1, pB, semB)

                wait_panel(pA, semA)
                gc = process(winAv, pA, gc)

                @pl.when(j2 < nwin_w)
                def _():
                    fetch(win2, pA, semA)

                def do_b(gcb):
                    wait_panel(pB, semB)
                    return process(winBv, pB, gcb)

                gc = lax.cond(j1 < nwin_w, do_b, lambda g: g, gc)
                return (gc, winAv + 2 * NW)

            gcnt, _ = lax.fori_loop(0, NPAIR, pair_body, (0, widv))
            rem = lax.min(gcnt, RB)
            lax.fori_loop(0, rem, lambda e, x: (drain_one(), x)[1], 0)

    return gather_kernel(user_ids, item_ids, ut_t, it_t, widtab)


def _tc_mlp(user_emb, item_emb, content, W_content, b_content, W1, b1, W2, b2):
    BLK = 2048
    cdim = content.shape[1]

    def body(ue_ref, ie_ref, c_ref, wc_ref, bc_ref, w1_ref, b1_ref, w2_ref,
             b2_ref, o_ref):
        c_emb = jnp.dot(c_ref[...], wc_ref[...],
                        preferred_element_type=jnp.float32) + bc_ref[...]
        h = jnp.dot(ue_ref[...], w1_ref[0:D, :],
                    preferred_element_type=jnp.float32)
        h = h + jnp.dot(ie_ref[...], w1_ref[D:2 * D, :],
                        preferred_element_type=jnp.float32)
        h = h + jnp.dot(c_emb, w1_ref[2 * D:3 * D, :],
                        preferred_element_type=jnp.float32)
        h = jnp.maximum(h + b1_ref[...], 0.0)
        o_ref[...] = jnp.dot(h, w2_ref[...],
                             preferred_element_type=jnp.float32) + b2_ref[...]

    full = lambda shape: pl.BlockSpec(shape, lambda i: (0, 0))
    out = pl.pallas_call(
        body,
        grid=(BATCH // BLK,),
        in_specs=[
            pl.BlockSpec((BLK, D), lambda i: (i, 0)),
            pl.BlockSpec((BLK, D), lambda i: (i, 0)),
            pl.BlockSpec((BLK, cdim), lambda i: (i, 0)),
            full((cdim, D)),
            full((1, D)),
            full((3 * D, D)),
            full((1, D)),
            full((D, 1)),
            full((1, 1)),
        ],
        out_specs=pl.BlockSpec((BLK, 1), lambda i: (i, 0)),
        out_shape=jax.ShapeDtypeStruct((BATCH, 1), jnp.float32),
    )(user_emb, item_emb, content, W_content, b_content, W1, b1, W2, b2)
    return out


def kernel(user_ids, item_ids, content, user_table, item_table, W_content,
           b_content, W1, b1, W2, b2):
    widtab = jnp.repeat(jnp.arange(NW, dtype=jnp.int32), L)
    user_emb, item_emb = _sc_scan_gather(
        user_ids.astype(jnp.int32), item_ids.astype(jnp.int32),
        jnp.transpose(user_table), jnp.transpose(item_table), widtab)
    out = _tc_mlp(user_emb, item_emb, content, W_content,
                  b_content.reshape(1, D), W1, b1.reshape(1, D), W2,
                  b2.reshape(1, 1))
    return out.reshape(-1)
